# Initial kernel scaffold; baseline (speedup 1.0000x reference)
import jax
import jax.numpy as jnp
from jax.experimental import pallas as pl


def kernel(data, edge_index, W, b, bn_gamma, bn_beta):
    raise NotImplementedError("write your pallas kernel here")


# speedup vs baseline: 17.7992x; 17.7992x over previous
import functools

import jax
import jax.numpy as jnp
from jax import lax
from jax.experimental import pallas as pl
from jax.experimental.pallas import tpu as pltpu
from jax.experimental.pallas import tpu_sc as plsc

N_NODES = 10000
N_EDGES = 320000
D = 128

NC = 2
NS = 16
NW = NC * NS
EPT = N_EDGES // NW
N_PAD = 10240
ROWS_PT = N_PAD // NS

PW = 5120
NPASS = 2


def _sc_degree_body(dst_hbm, deg_part_hbm, idx_v, hist_v, res_v):
    c = lax.axis_index("c")
    s = lax.axis_index("s")
    wid = s * NC + c
    base = wid * EPT
    pltpu.sync_copy(dst_hbm.at[pl.ds(base, EPT)], idx_v)

    lanes = jnp.arange(16, dtype=jnp.int32)
    ones = jnp.ones((16,), jnp.float32)
    zeros = jnp.zeros((16,), jnp.float32)

    for p in range(NPASS):
        lo = p * PW

        @pl.loop(0, 16 * PW // 16)
        def _zero(col):
            hist_v[pl.ds(col * 16, 16)] = zeros

        @pl.loop(0, EPT // 16)
        def _scan(e):
            idx16 = idx_v[pl.ds(e * 16, 16)]
            local = idx16 - lo
            mask = (local >= 0) & (local < PW)
            localc = jnp.where(mask, local, 0)
            plsc.addupdate_scatter(hist_v, [lanes * PW + localc], ones,
                                   mask=mask)

        @pl.loop(0, PW // 16)
        def _reduce(col):
            acc = hist_v[pl.ds(col * 16, 16)]
            for r in range(1, 16):
                acc = acc + hist_v[pl.ds(r * PW + col * 16, 16)]
            res_v[pl.ds(col * 16, 16)] = acc

        pltpu.sync_copy(res_v, deg_part_hbm.at[wid, pl.ds(lo, PW)])


_sc_degree = functools.partial(
    pl.kernel,
    out_type=jax.ShapeDtypeStruct((NW, NPASS * PW), jnp.float32),
    mesh=plsc.VectorSubcoreMesh(core_axis_name="c", subcore_axis_name="s",
                                num_cores=NC, num_subcores=NS),
    scratch_types=[
        pltpu.VMEM((EPT,), jnp.int32),
        pltpu.VMEM((16 * PW,), jnp.float32),
        pltpu.VMEM((PW,), jnp.float32),
    ],
    compiler_params=pltpu.CompilerParams(needs_layout_passes=False),
)(_sc_degree_body)


def _tc_prep_body(deg_ref, data_ref, w_ref, y_ref, dinv_ref):
    deg = jnp.sum(deg_ref[...], axis=0)[:N_NODES] + 1.0
    dinv = lax.rsqrt(deg)[:, None]
    xw = jnp.dot(data_ref[...], w_ref[...],
                 preferred_element_type=jnp.float32)
    y_ref[...] = xw * dinv
    dinv_ref[...] = dinv


def _tc_prep(deg_part, data, W):
    return pl.pallas_call(
        _tc_prep_body,
        out_shape=[
            jax.ShapeDtypeStruct((N_NODES, D), jnp.float32),
            jax.ShapeDtypeStruct((N_NODES, 1), jnp.float32),
        ],
    )(deg_part, data, W)


G = 80
NCHUNK = EPT // G
ZR = 125


def _sc_scatter_body(src_hbm, dst_hbm, y_hbm, zeros_hbm, part_hbm,
                     sidx_v, didx_v, rows_v, acc_sh):
    c = lax.axis_index("c")
    s = lax.axis_index("s")
    wid = s * NC + c
    base = wid * EPT

    pltpu.sync_copy(zeros_hbm, acc_sh.at[pl.ds(s * ROWS_PT, ROWS_PT), :])

    plsc.subcore_barrier()

    @pl.loop(0, NCHUNK)
    def _chunk(i):
        off = base + i * G
        pltpu.sync_copy(src_hbm.at[pl.ds(off, G)], sidx_v)
        pltpu.sync_copy(dst_hbm.at[pl.ds(off, G)], didx_v)
        pltpu.sync_copy(y_hbm.at[sidx_v], rows_v)
        pltpu.sync_copy(rows_v, acc_sh.at[didx_v], add=True)

    plsc.subcore_barrier()

    pltpu.sync_copy(acc_sh.at[pl.ds(s * ROWS_PT, ROWS_PT), :],
                    part_hbm.at[c, pl.ds(s * ROWS_PT, ROWS_PT), :])


_sc_scatter = functools.partial(
    pl.kernel,
    out_type=jax.ShapeDtypeStruct((NC, N_PAD, D), jnp.float32),
    mesh=plsc.VectorSubcoreMesh(core_axis_name="c", subcore_axis_name="s",
                                num_cores=NC, num_subcores=NS),
    scratch_types=[
        pltpu.VMEM((G,), jnp.int32),
        pltpu.VMEM((G,), jnp.int32),
        pltpu.VMEM((G, D), jnp.float32),
        pltpu.VMEM_SHARED((N_PAD, D), jnp.float32),
    ],
)(_sc_scatter_body)


def _tc_finish_body(part_ref, y_ref, dinv_ref, b_ref, g_ref, beta_ref,
                    o_ref):
    s = part_ref[0, :N_NODES] + part_ref[1, :N_NODES] + y_ref[...]
    pre = s * dinv_ref[...] + b_ref[...]
    r = jnp.maximum(pre, 0.0)
    mean = jnp.mean(r, axis=0, keepdims=True)
    var = jnp.mean((r - mean) ** 2, axis=0, keepdims=True)
    o_ref[...] = (r - mean) / jnp.sqrt(var + 1e-5) * g_ref[...] + beta_ref[...]


def _tc_finish(part, y, dinv, b, g, beta):
    return pl.pallas_call(
        _tc_finish_body,
        out_shape=jax.ShapeDtypeStruct((N_NODES, D), jnp.float32),
    )(part, y, dinv, b, g, beta)


def kernel(data, edge_index, W, b, bn_gamma, bn_beta):
    src = edge_index[0].astype(jnp.int32)
    dst = edge_index[1].astype(jnp.int32)
    deg_part = _sc_degree(dst)
    y, dinv = _tc_prep(deg_part, data, W)
    zeros_rows = jnp.zeros((ROWS_PT, D), jnp.float32)
    part = _sc_scatter(src, dst, y, zeros_rows)
    return _tc_finish(part, y, dinv, b.reshape(1, D),
                      bn_gamma.reshape(1, D), bn_beta.reshape(1, D))
